# baseline (speedup 1.0000x reference)
import functools

import jax
import jax.numpy as jnp
from jax import lax
from jax.experimental import pallas as pl
from jax.experimental.pallas import tpu as pltpu
from jax.experimental.pallas import tpu_sc as plsc

NC = 2
NS = 16
NW = NC * NS
CHUNK = 128
ROWS = 512
NBUF = 2
DI = 2 * NBUF
L = 16

def _sc_mesh():
    return plsc.VectorSubcoreMesh(
        core_axis_name="c", subcore_axis_name="s",
        num_cores=NC, num_subcores=NS)


def _make_sc_msg(n_pad, d, e_pad):
    t_chunks = e_pad // (NS * CHUNK)
    assert t_chunks % DI == 0 and t_chunks >= 4 * DI
    m1 = min(DI, t_chunks // 2)
    m0 = t_chunks - m1
    stripe = n_pad // NS
    z_iters = stripe // CHUNK

    @functools.partial(
        pl.kernel,
        out_type=jax.ShapeDtypeStruct((NC, n_pad, d), jnp.float32),
        mesh=_sc_mesh(),
        scratch_types=[
            pltpu.VMEM((DI, CHUNK), jnp.int32),
            pltpu.VMEM((DI, CHUNK), jnp.int32),
            pltpu.VMEM((NBUF, CHUNK, d), jnp.float32),
            pltpu.VMEM_SHARED((n_pad, d), jnp.float32),
            [pltpu.SemaphoreType.DMA] * DI,
            [pltpu.SemaphoreType.DMA] * NBUF,
            pltpu.SemaphoreType.DMA,
        ],
    )
    def msg(src_hbm, dst_hbm, g_hbm, zeros_hbm, out_hbm,
            idx_s, idx_d, rows, acc_sh, sem_i, sem_g, sem_z):
        c = lax.axis_index("c")
        t = lax.axis_index("s")
        is0 = c == 0
        m = jnp.where(is0, m0, m1)
        base = jnp.where(is0, t * m0, NS * m0 + t * m1)
        n_groups = jnp.where(is0, m0 // DI, m1 // DI)

        if True:
            for k in range(z_iters):
                pltpu.async_copy(
                    zeros_hbm, acc_sh.at[pl.ds(t * stripe + k * CHUNK, CHUNK)],
                    sem_z)

            def load_idx(j, sl):
                pltpu.async_copy(src_hbm.at[base + j], idx_s.at[sl], sem_i[sl])
                pltpu.async_copy(dst_hbm.at[base + j], idx_d.at[sl], sem_i[sl])

            def wait_idx(j, sl):
                pltpu.make_async_copy(
                    src_hbm.at[base + j], idx_s.at[sl], sem_i[sl]).wait()
                pltpu.make_async_copy(
                    dst_hbm.at[base + j], idx_d.at[sl], sem_i[sl]).wait()

            def start_gather(sl, b):
                pltpu.async_copy(g_hbm.at[idx_s.at[sl]], rows.at[b], sem_g[b])

            def wait_gather(sl, b):
                pltpu.make_async_copy(
                    g_hbm.at[idx_s.at[sl]], rows.at[b], sem_g[b]).wait()

            with jax.named_scope("sc_prologue"):
                for j in range(DI):
                    load_idx(j, j)
                for b in range(NBUF):
                    wait_idx(b, b)
                    start_gather(b, b)
                for k in range(z_iters):
                    pltpu.make_async_copy(
                        zeros_hbm,
                        acc_sh.at[pl.ds(t * stripe + k * CHUNK, CHUNK)],
                        sem_z).wait()
                plsc.subcore_barrier()

            def group(kk, carry):
                for u in range(DI):
                    i = DI * kk + u
                    b = u % NBUF
                    wait_gather(u, b)
                    pltpu.sync_copy(rows.at[b], acc_sh.at[idx_d.at[u]],
                                    add=True)
                    j = i + NBUF

                    @pl.when(j < m)
                    def _():
                        sl = (u + NBUF) % DI
                        wait_idx(j, sl)
                        start_gather(sl, b)

                    jj = i + DI

                    @pl.when(jj < m)
                    def _():
                        load_idx(jj, u)
                return carry

            with jax.named_scope("sc_edges"):
                lax.fori_loop(0, n_groups, group, 0)
                plsc.subcore_barrier()
            with jax.named_scope("sc_dump"):
                for k in range(z_iters):
                    off = t * stripe + k * CHUNK
                    pltpu.sync_copy(acc_sh.at[pl.ds(off, CHUNK)],
                                    out_hbm.at[c].at[pl.ds(off, CHUNK)])

    return msg


DEG_W = 128
DEG_DI = 8
DEG_NS = 4


def _make_sc_deg(n_pad, e_pad):
    t_chunks = e_pad // (NS * CHUNK)
    assert t_chunks % DEG_DI == 0 and t_chunks >= 2 * DEG_DI
    stripe = n_pad // NS

    @functools.partial(
        pl.kernel,
        out_type=jax.ShapeDtypeStruct((n_pad, DEG_W), jnp.float32),
        mesh=_sc_mesh(),
        scratch_types=[
            pltpu.VMEM((DEG_DI, CHUNK), jnp.int32),
            pltpu.VMEM((CHUNK, DEG_W), jnp.float32),
            pltpu.VMEM_SHARED((n_pad, DEG_W), jnp.float32),
            [pltpu.SemaphoreType.DMA] * DEG_DI,
            [pltpu.SemaphoreType.DMA] * DEG_NS,
            pltpu.SemaphoreType.DMA,
        ],
    )
    def deg(dst_hbm, zeros_hbm, ones_hbm, out_hbm,
            idx_d, ones_v, acc_sh, sem_i, sem_s, sem_z):
        c = lax.axis_index("c")
        t = lax.axis_index("s")

        @pl.when(c == 0)
        def _body():
            base = t * t_chunks
            pltpu.async_copy(
                zeros_hbm, acc_sh.at[pl.ds(t * stripe, stripe)], sem_z)
            pltpu.async_copy(ones_hbm, ones_v, sem_z)

            def load_idx(j, sl):
                pltpu.async_copy(dst_hbm.at[base + j], idx_d.at[sl], sem_i[sl])

            def wait_idx(j, sl):
                pltpu.make_async_copy(
                    dst_hbm.at[base + j], idx_d.at[sl], sem_i[sl]).wait()

            def start_scatter(sl, su):
                pltpu.async_copy(ones_v, acc_sh.at[idx_d.at[sl]], sem_s[su],
                                 add=True)

            def wait_scatter(sl, su):
                pltpu.make_async_copy(
                    ones_v, acc_sh.at[idx_d.at[sl]], sem_s[su]).wait()

            for j in range(DEG_DI):
                load_idx(j, j)
            pltpu.make_async_copy(
                zeros_hbm, acc_sh.at[pl.ds(t * stripe, stripe)], sem_z).wait()
            pltpu.make_async_copy(ones_hbm, ones_v, sem_z).wait()
            plsc.subcore_barrier()

            def group(kk, carry):
                for u in range(DEG_DI):
                    i = DEG_DI * kk + u
                    su = u % DEG_NS
                    wait_idx(i, u)
                    jj = i + DEG_NS

                    @pl.when((i >= DEG_NS) & (jj < t_chunks))
                    def _():
                        load_idx(jj, (u + DEG_NS) % DEG_DI)

                    pltpu.sync_copy(ones_v, acc_sh.at[idx_d.at[u]], add=True)
                return carry

            lax.fori_loop(0, t_chunks // DEG_DI, group, 0)
            plsc.subcore_barrier()
            pltpu.sync_copy(acc_sh.at[pl.ds(t * stripe, stripe)],
                            out_hbm.at[pl.ds(t * stripe, stripe)])

    return deg


def _row_ids(i):
    return i * ROWS + lax.broadcasted_iota(jnp.int32, (ROWS, 1), 0)


def _tc_dinv_pre(deg2, x, w, n, n_pad, d):
    def body(dg_ref, x_ref, w_ref, dinv_ref, g_ref):
        deg = dg_ref[:, 0:1] + 1.0
        dinv = jnp.where(_row_ids(pl.program_id(0)) < n, lax.rsqrt(deg), 0.0)
        dinv_ref[...] = dinv
        g_ref[...] = dinv * jnp.dot(x_ref[...], w_ref[...],
                                    preferred_element_type=jnp.float32)

    return pl.pallas_call(
        body,
        grid=(n_pad // ROWS,),
        in_specs=[
            pl.BlockSpec((ROWS, DEG_W), lambda i: (i, 0)),
            pl.BlockSpec((ROWS, d), lambda i: (i, 0)),
            pl.BlockSpec((d, d), lambda i: (0, 0)),
        ],
        out_specs=[
            pl.BlockSpec((ROWS, 1), lambda i: (i, 0)),
            pl.BlockSpec((ROWS, d), lambda i: (i, 0)),
        ],
        out_shape=[
            jax.ShapeDtypeStruct((n_pad, 1), jnp.float32),
            jax.ShapeDtypeStruct((n_pad, d), jnp.float32),
        ],
    )(deg2, x, w)


def _layer_h(s0_ref, s1_ref, g_ref, dinv_ref, b_ref, n, i):
    t = dinv_ref[...] * (s0_ref[...] + s1_ref[...] + g_ref[...]) + b_ref[...]
    nrm = jnp.sqrt(jnp.sum(t * t, axis=1, keepdims=True))
    h = jnp.maximum(t / jnp.maximum(nrm, 1e-12), 0.0)
    return jnp.where(_row_ids(i) < n, h, 0.0)


def _tc_post_pre(s, g, dinv, b, w_next, n, n_pad, d):
    def body(s0_ref, s1_ref, g_ref, dinv_ref, b_ref, w_ref, o_ref):
        h = _layer_h(s0_ref, s1_ref, g_ref, dinv_ref, b_ref, n,
                     pl.program_id(0))
        o_ref[...] = dinv_ref[...] * jnp.dot(
            h, w_ref[...], preferred_element_type=jnp.float32)

    return pl.pallas_call(
        body,
        grid=(n_pad // ROWS,),
        in_specs=[
            pl.BlockSpec((None, ROWS, d), lambda i: (0, i, 0)),
            pl.BlockSpec((None, ROWS, d), lambda i: (1, i, 0)),
            pl.BlockSpec((ROWS, d), lambda i: (i, 0)),
            pl.BlockSpec((ROWS, 1), lambda i: (i, 0)),
            pl.BlockSpec((1, d), lambda i: (0, 0)),
            pl.BlockSpec((d, d), lambda i: (0, 0)),
        ],
        out_specs=pl.BlockSpec((ROWS, d), lambda i: (i, 0)),
        out_shape=jax.ShapeDtypeStruct((n_pad, d), jnp.float32),
    )(s, s, g, dinv, b, w_next)


def _tc_post_pool_head(s, g, dinv, b, w1, b1, w2p, b2p, n, n_pad, d):
    ng = n_pad // ROWS

    def body(s0_ref, s1_ref, g_ref, dinv_ref, b_ref, w1_ref, b1_ref, w2_ref,
             b2_ref, o_ref, acc):
        i = pl.program_id(0)
        h = _layer_h(s0_ref, s1_ref, g_ref, dinv_ref, b_ref, n, i)

        @pl.when(i == 0)
        def _():
            acc[...] = jnp.zeros_like(acc)

        acc[...] += jnp.sum(h, axis=0, keepdims=True)

        @pl.when(i == ng - 1)
        def _():
            z = jnp.maximum(
                jnp.dot(acc[...], w1_ref[...],
                        preferred_element_type=jnp.float32) + b1_ref[...], 0.0)
            o_ref[...] = jnp.dot(
                z, w2_ref[...], preferred_element_type=jnp.float32) + b2_ref[...]

    return pl.pallas_call(
        body,
        grid=(ng,),
        in_specs=[
            pl.BlockSpec((None, ROWS, d), lambda i: (0, i, 0)),
            pl.BlockSpec((None, ROWS, d), lambda i: (1, i, 0)),
            pl.BlockSpec((ROWS, d), lambda i: (i, 0)),
            pl.BlockSpec((ROWS, 1), lambda i: (i, 0)),
            pl.BlockSpec((1, d), lambda i: (0, 0)),
            pl.BlockSpec((d, d), lambda i: (0, 0)),
            pl.BlockSpec((1, d), lambda i: (0, 0)),
            pl.BlockSpec((d, d), lambda i: (0, 0)),
            pl.BlockSpec((1, d), lambda i: (0, 0)),
        ],
        out_specs=pl.BlockSpec((1, d), lambda i: (0, 0)),
        out_shape=jax.ShapeDtypeStruct((1, d), jnp.float32),
        scratch_shapes=[pltpu.VMEM((1, d), jnp.float32)],
    )(s, s, g, dinv, b, w1, b1, w2p, b2p)


def kernel(x, edge_index, batch, W0, b0, W1, b1, W2, b2,
           lin1_W, lin1_b, lin2_W, lin2_b):
    n, d = x.shape
    e = edge_index.shape[1]
    c_out = lin2_W.shape[1]

    n_pad = ((n + 1 + 2047) // 2048) * 2048
    e_quant = NS * CHUNK * DEG_DI
    e_pad = ((e + e_quant - 1) // e_quant) * e_quant

    src = jnp.pad(edge_index[0].astype(jnp.int32), (0, e_pad - e),
                  constant_values=n).reshape(e_pad // CHUNK, CHUNK)
    dst = jnp.pad(edge_index[1].astype(jnp.int32), (0, e_pad - e),
                  constant_values=n).reshape(e_pad // CHUNK, CHUNK)
    x_pad = jnp.pad(x.astype(jnp.float32), ((0, n_pad - n), (0, 0)))
    zeros_blk = jnp.zeros((CHUNK, d), jnp.float32)
    zeros_stripe = jnp.zeros((n_pad // NS, DEG_W), jnp.float32)
    ones_blk = jnp.ones((CHUNK, DEG_W), jnp.float32)

    sc_msg = _make_sc_msg(n_pad, d, e_pad)
    sc_deg = _make_sc_deg(n_pad, e_pad)

    deg2 = sc_deg(dst, zeros_stripe, ones_blk)
    dinv, g = _tc_dinv_pre(deg2, x_pad, W0, n, n_pad, d)

    s = sc_msg(src, dst, g, zeros_blk)
    g = _tc_post_pre(s, g, dinv, b0.reshape(1, d), W1, n, n_pad, d)
    s = sc_msg(src, dst, g, zeros_blk)
    g = _tc_post_pre(s, g, dinv, b1.reshape(1, d), W2, n, n_pad, d)
    s = sc_msg(src, dst, g, zeros_blk)

    w2p = jnp.pad(lin2_W, ((0, 0), (0, d - c_out)))
    b2p = jnp.pad(lin2_b, (0, d - c_out)).reshape(1, d)
    out_full = _tc_post_pool_head(
        s, g, dinv, b2.reshape(1, d), lin1_W, lin1_b.reshape(1, d),
        w2p, b2p, n, n_pad, d)
    return out_full[:, :c_out]


# speedup vs baseline: 1.0046x; 1.0046x over previous
import functools

import jax
import jax.numpy as jnp
from jax import lax
from jax.experimental import pallas as pl
from jax.experimental.pallas import tpu as pltpu
from jax.experimental.pallas import tpu_sc as plsc

NC = 2
NS = 16
NW = NC * NS
CHUNK = 128
ROWS = 512
NBUF = 2
DI = 2 * NBUF
L = 16

def _sc_mesh():
    return plsc.VectorSubcoreMesh(
        core_axis_name="c", subcore_axis_name="s",
        num_cores=NC, num_subcores=NS)


def _make_sc_msg(n_pad, d, e_pad):
    t_chunks = e_pad // (NS * CHUNK)
    assert t_chunks % DI == 0 and t_chunks >= 4 * DI
    m1 = min(max((int(t_chunks * 0.05) // DI) * DI, DI), t_chunks // 2)
    m0 = t_chunks - m1
    stripe = n_pad // NS
    z_iters = stripe // CHUNK

    @functools.partial(
        pl.kernel,
        out_type=jax.ShapeDtypeStruct((NC, n_pad, d), jnp.float32),
        mesh=_sc_mesh(),
        scratch_types=[
            pltpu.VMEM((DI, CHUNK), jnp.int32),
            pltpu.VMEM((DI, CHUNK), jnp.int32),
            pltpu.VMEM((NBUF, CHUNK, d), jnp.float32),
            pltpu.VMEM_SHARED((n_pad, d), jnp.float32),
            [pltpu.SemaphoreType.DMA] * DI,
            [pltpu.SemaphoreType.DMA] * NBUF,
            pltpu.SemaphoreType.DMA,
        ],
    )
    def msg(src_hbm, dst_hbm, g_hbm, zeros_hbm, out_hbm,
            idx_s, idx_d, rows, acc_sh, sem_i, sem_g, sem_z):
        c = lax.axis_index("c")
        t = lax.axis_index("s")
        is0 = c == 0
        m = jnp.where(is0, m0, m1)
        base = jnp.where(is0, t * m0, NS * m0 + t * m1)
        n_groups = jnp.where(is0, m0 // DI, m1 // DI)

        if True:
            for k in range(z_iters):
                pltpu.async_copy(
                    zeros_hbm, acc_sh.at[pl.ds(t * stripe + k * CHUNK, CHUNK)],
                    sem_z)

            def load_idx(j, sl):
                pltpu.async_copy(src_hbm.at[base + j], idx_s.at[sl], sem_i[sl])
                pltpu.async_copy(dst_hbm.at[base + j], idx_d.at[sl], sem_i[sl])

            def wait_idx(j, sl):
                pltpu.make_async_copy(
                    src_hbm.at[base + j], idx_s.at[sl], sem_i[sl]).wait()
                pltpu.make_async_copy(
                    dst_hbm.at[base + j], idx_d.at[sl], sem_i[sl]).wait()

            def start_gather(sl, b):
                pltpu.async_copy(g_hbm.at[idx_s.at[sl]], rows.at[b], sem_g[b])

            def wait_gather(sl, b):
                pltpu.make_async_copy(
                    g_hbm.at[idx_s.at[sl]], rows.at[b], sem_g[b]).wait()

            with jax.named_scope("sc_prologue"):
                for j in range(DI):
                    load_idx(j, j)
                for b in range(NBUF):
                    wait_idx(b, b)
                    start_gather(b, b)
                for k in range(z_iters):
                    pltpu.make_async_copy(
                        zeros_hbm,
                        acc_sh.at[pl.ds(t * stripe + k * CHUNK, CHUNK)],
                        sem_z).wait()
                plsc.subcore_barrier()

            def group(kk, carry):
                for u in range(DI):
                    i = DI * kk + u
                    b = u % NBUF
                    wait_gather(u, b)
                    pltpu.sync_copy(rows.at[b], acc_sh.at[idx_d.at[u]],
                                    add=True)
                    j = i + NBUF

                    @pl.when(j < m)
                    def _():
                        sl = (u + NBUF) % DI
                        wait_idx(j, sl)
                        start_gather(sl, b)

                    jj = i + DI

                    @pl.when(jj < m)
                    def _():
                        load_idx(jj, u)
                return carry

            with jax.named_scope("sc_edges"):
                lax.fori_loop(0, n_groups, group, 0)
                plsc.subcore_barrier()
            with jax.named_scope("sc_dump"):
                for k in range(z_iters):
                    off = t * stripe + k * CHUNK
                    pltpu.sync_copy(acc_sh.at[pl.ds(off, CHUNK)],
                                    out_hbm.at[c].at[pl.ds(off, CHUNK)])

    return msg


DEG_W = 128
DEG_DI = 8
DEG_NS = 4


def _make_sc_deg(n_pad, e_pad):
    t_chunks = e_pad // (NS * CHUNK)
    assert t_chunks % DEG_DI == 0 and t_chunks >= 2 * DEG_DI
    stripe = n_pad // NS

    @functools.partial(
        pl.kernel,
        out_type=jax.ShapeDtypeStruct((n_pad, DEG_W), jnp.float32),
        mesh=_sc_mesh(),
        scratch_types=[
            pltpu.VMEM((DEG_DI, CHUNK), jnp.int32),
            pltpu.VMEM((CHUNK, DEG_W), jnp.float32),
            pltpu.VMEM_SHARED((n_pad, DEG_W), jnp.float32),
            [pltpu.SemaphoreType.DMA] * DEG_DI,
            [pltpu.SemaphoreType.DMA] * DEG_NS,
            pltpu.SemaphoreType.DMA,
        ],
    )
    def deg(dst_hbm, zeros_hbm, ones_hbm, out_hbm,
            idx_d, ones_v, acc_sh, sem_i, sem_s, sem_z):
        c = lax.axis_index("c")
        t = lax.axis_index("s")

        @pl.when(c == 0)
        def _body():
            base = t * t_chunks
            pltpu.async_copy(
                zeros_hbm, acc_sh.at[pl.ds(t * stripe, stripe)], sem_z)
            pltpu.async_copy(ones_hbm, ones_v, sem_z)

            def load_idx(j, sl):
                pltpu.async_copy(dst_hbm.at[base + j], idx_d.at[sl], sem_i[sl])

            def wait_idx(j, sl):
                pltpu.make_async_copy(
                    dst_hbm.at[base + j], idx_d.at[sl], sem_i[sl]).wait()

            def start_scatter(sl, su):
                pltpu.async_copy(ones_v, acc_sh.at[idx_d.at[sl]], sem_s[su],
                                 add=True)

            def wait_scatter(sl, su):
                pltpu.make_async_copy(
                    ones_v, acc_sh.at[idx_d.at[sl]], sem_s[su]).wait()

            for j in range(DEG_DI):
                load_idx(j, j)
            pltpu.make_async_copy(
                zeros_hbm, acc_sh.at[pl.ds(t * stripe, stripe)], sem_z).wait()
            pltpu.make_async_copy(ones_hbm, ones_v, sem_z).wait()
            plsc.subcore_barrier()

            def group(kk, carry):
                for u in range(DEG_DI):
                    i = DEG_DI * kk + u
                    su = u % DEG_NS
                    wait_idx(i, u)
                    jj = i + DEG_NS

                    @pl.when((i >= DEG_NS) & (jj < t_chunks))
                    def _():
                        load_idx(jj, (u + DEG_NS) % DEG_DI)

                    pltpu.sync_copy(ones_v, acc_sh.at[idx_d.at[u]], add=True)
                return carry

            lax.fori_loop(0, t_chunks // DEG_DI, group, 0)
            plsc.subcore_barrier()
            pltpu.sync_copy(acc_sh.at[pl.ds(t * stripe, stripe)],
                            out_hbm.at[pl.ds(t * stripe, stripe)])

    return deg


def _row_ids(i):
    return i * ROWS + lax.broadcasted_iota(jnp.int32, (ROWS, 1), 0)


def _tc_dinv_pre(deg2, x, w, n, n_pad, d):
    def body(dg_ref, x_ref, w_ref, dinv_ref, g_ref):
        deg = dg_ref[:, 0:1] + 1.0
        dinv = jnp.where(_row_ids(pl.program_id(0)) < n, lax.rsqrt(deg), 0.0)
        dinv_ref[...] = dinv
        g_ref[...] = dinv * jnp.dot(x_ref[...], w_ref[...],
                                    preferred_element_type=jnp.float32)

    return pl.pallas_call(
        body,
        grid=(n_pad // ROWS,),
        in_specs=[
            pl.BlockSpec((ROWS, DEG_W), lambda i: (i, 0)),
            pl.BlockSpec((ROWS, d), lambda i: (i, 0)),
            pl.BlockSpec((d, d), lambda i: (0, 0)),
        ],
        out_specs=[
            pl.BlockSpec((ROWS, 1), lambda i: (i, 0)),
            pl.BlockSpec((ROWS, d), lambda i: (i, 0)),
        ],
        out_shape=[
            jax.ShapeDtypeStruct((n_pad, 1), jnp.float32),
            jax.ShapeDtypeStruct((n_pad, d), jnp.float32),
        ],
    )(deg2, x, w)


def _layer_h(s0_ref, s1_ref, g_ref, dinv_ref, b_ref, n, i):
    t = dinv_ref[...] * (s0_ref[...] + s1_ref[...] + g_ref[...]) + b_ref[...]
    nrm = jnp.sqrt(jnp.sum(t * t, axis=1, keepdims=True))
    h = jnp.maximum(t / jnp.maximum(nrm, 1e-12), 0.0)
    return jnp.where(_row_ids(i) < n, h, 0.0)


def _tc_post_pre(s, g, dinv, b, w_next, n, n_pad, d):
    def body(s0_ref, s1_ref, g_ref, dinv_ref, b_ref, w_ref, o_ref):
        h = _layer_h(s0_ref, s1_ref, g_ref, dinv_ref, b_ref, n,
                     pl.program_id(0))
        o_ref[...] = dinv_ref[...] * jnp.dot(
            h, w_ref[...], preferred_element_type=jnp.float32)

    return pl.pallas_call(
        body,
        grid=(n_pad // ROWS,),
        in_specs=[
            pl.BlockSpec((None, ROWS, d), lambda i: (0, i, 0)),
            pl.BlockSpec((None, ROWS, d), lambda i: (1, i, 0)),
            pl.BlockSpec((ROWS, d), lambda i: (i, 0)),
            pl.BlockSpec((ROWS, 1), lambda i: (i, 0)),
            pl.BlockSpec((1, d), lambda i: (0, 0)),
            pl.BlockSpec((d, d), lambda i: (0, 0)),
        ],
        out_specs=pl.BlockSpec((ROWS, d), lambda i: (i, 0)),
        out_shape=jax.ShapeDtypeStruct((n_pad, d), jnp.float32),
    )(s, s, g, dinv, b, w_next)


def _tc_post_pool_head(s, g, dinv, b, w1, b1, w2p, b2p, n, n_pad, d):
    ng = n_pad // ROWS

    def body(s0_ref, s1_ref, g_ref, dinv_ref, b_ref, w1_ref, b1_ref, w2_ref,
             b2_ref, o_ref, acc):
        i = pl.program_id(0)
        h = _layer_h(s0_ref, s1_ref, g_ref, dinv_ref, b_ref, n, i)

        @pl.when(i == 0)
        def _():
            acc[...] = jnp.zeros_like(acc)

        acc[...] += jnp.sum(h, axis=0, keepdims=True)

        @pl.when(i == ng - 1)
        def _():
            z = jnp.maximum(
                jnp.dot(acc[...], w1_ref[...],
                        preferred_element_type=jnp.float32) + b1_ref[...], 0.0)
            o_ref[...] = jnp.dot(
                z, w2_ref[...], preferred_element_type=jnp.float32) + b2_ref[...]

    return pl.pallas_call(
        body,
        grid=(ng,),
        in_specs=[
            pl.BlockSpec((None, ROWS, d), lambda i: (0, i, 0)),
            pl.BlockSpec((None, ROWS, d), lambda i: (1, i, 0)),
            pl.BlockSpec((ROWS, d), lambda i: (i, 0)),
            pl.BlockSpec((ROWS, 1), lambda i: (i, 0)),
            pl.BlockSpec((1, d), lambda i: (0, 0)),
            pl.BlockSpec((d, d), lambda i: (0, 0)),
            pl.BlockSpec((1, d), lambda i: (0, 0)),
            pl.BlockSpec((d, d), lambda i: (0, 0)),
            pl.BlockSpec((1, d), lambda i: (0, 0)),
        ],
        out_specs=pl.BlockSpec((1, d), lambda i: (0, 0)),
        out_shape=jax.ShapeDtypeStruct((1, d), jnp.float32),
        scratch_shapes=[pltpu.VMEM((1, d), jnp.float32)],
    )(s, s, g, dinv, b, w1, b1, w2p, b2p)


def kernel(x, edge_index, batch, W0, b0, W1, b1, W2, b2,
           lin1_W, lin1_b, lin2_W, lin2_b):
    n, d = x.shape
    e = edge_index.shape[1]
    c_out = lin2_W.shape[1]

    n_pad = ((n + 1 + 2047) // 2048) * 2048
    e_quant = NS * CHUNK * DEG_DI
    e_pad = ((e + e_quant - 1) // e_quant) * e_quant

    src = jnp.pad(edge_index[0].astype(jnp.int32), (0, e_pad - e),
                  constant_values=n).reshape(e_pad // CHUNK, CHUNK)
    dst = jnp.pad(edge_index[1].astype(jnp.int32), (0, e_pad - e),
                  constant_values=n).reshape(e_pad // CHUNK, CHUNK)
    x_pad = jnp.pad(x.astype(jnp.float32), ((0, n_pad - n), (0, 0)))
    zeros_blk = jnp.zeros((CHUNK, d), jnp.float32)
    zeros_stripe = jnp.zeros((n_pad // NS, DEG_W), jnp.float32)
    ones_blk = jnp.ones((CHUNK, DEG_W), jnp.float32)

    sc_msg = _make_sc_msg(n_pad, d, e_pad)
    sc_deg = _make_sc_deg(n_pad, e_pad)

    deg2 = sc_deg(dst, zeros_stripe, ones_blk)
    dinv, g = _tc_dinv_pre(deg2, x_pad, W0, n, n_pad, d)

    s = sc_msg(src, dst, g, zeros_blk)
    g = _tc_post_pre(s, g, dinv, b0.reshape(1, d), W1, n, n_pad, d)
    s = sc_msg(src, dst, g, zeros_blk)
    g = _tc_post_pre(s, g, dinv, b1.reshape(1, d), W2, n, n_pad, d)
    s = sc_msg(src, dst, g, zeros_blk)

    w2p = jnp.pad(lin2_W, ((0, 0), (0, d - c_out)))
    b2p = jnp.pad(lin2_b, (0, d - c_out)).reshape(1, d)
    out_full = _tc_post_pool_head(
        s, g, dinv, b2.reshape(1, d), lin1_W, lin1_b.reshape(1, d),
        w2p, b2p, n, n_pad, d)
    return out_full[:, :c_out]
